# Initial kernel scaffold; baseline (speedup 1.0000x reference)
import jax
import jax.numpy as jnp
from jax.experimental import pallas as pl


def kernel(indices, table):
    raise NotImplementedError("write your pallas kernel here")


# speedup vs baseline: 5.1820x; 5.1820x over previous
import functools

import jax
import jax.numpy as jnp
from jax import lax
from jax.experimental import pallas as pl
from jax.experimental.pallas import tpu as pltpu
from jax.experimental.pallas import tpu_sc as plsc

DIM = 128
NC, NS = 2, 16
NW = NC * NS
CHUNK = 128


def _make_gather(rows):
    rows_per_w = rows // NW
    chunks = rows_per_w // CHUNK
    mesh = plsc.VectorSubcoreMesh(core_axis_name="c", subcore_axis_name="s")

    @functools.partial(
        pl.kernel,
        mesh=mesh,
        out_type=jax.ShapeDtypeStruct((rows, DIM), jnp.float32),
        scratch_types=[
            pltpu.VMEM((CHUNK,), jnp.int32),
            pltpu.VMEM((CHUNK, DIM), jnp.float32),
            pltpu.SemaphoreType.DMA,
        ],
    )
    def gather_kernel(idx_hbm, table_hbm, out_hbm, idx_v, rows_v, sem):
        wid = lax.axis_index("s") * NC + lax.axis_index("c")
        base = wid * rows_per_w

        def body(g, carry):
            off = base + g * CHUNK
            pltpu.sync_copy(idx_hbm.at[pl.ds(off, CHUNK)], idx_v)
            pltpu.async_copy(table_hbm.at[idx_v], rows_v, sem).wait()
            pltpu.sync_copy(rows_v, out_hbm.at[pl.ds(off, CHUNK)])
            return carry

        lax.fori_loop(0, chunks, body, 0)

    return gather_kernel


def kernel(indices, table):
    batch, seq = indices.shape
    rows = batch * seq
    out = _make_gather(rows)(indices.reshape(rows), table)
    return out.reshape(batch, seq, DIM)


# speedup vs baseline: 9.2638x; 1.7877x over previous
import functools

import jax
import jax.numpy as jnp
from jax import lax
from jax.experimental import pallas as pl
from jax.experimental.pallas import tpu as pltpu
from jax.experimental.pallas import tpu_sc as plsc

DIM = 128
NC, NS = 2, 16
NW = NC * NS
CHUNK = 128
NBUF = 4


def _make_gather(rows):
    rows_per_w = rows // NW
    chunks = rows_per_w // CHUNK
    groups = chunks // NBUF
    mesh = plsc.VectorSubcoreMesh(core_axis_name="c", subcore_axis_name="s")

    scratch = (
        [pltpu.VMEM((CHUNK,), jnp.int32) for _ in range(NBUF)]
        + [pltpu.VMEM((CHUNK, DIM), jnp.float32) for _ in range(NBUF)]
        + [pltpu.SemaphoreType.DMA for _ in range(2 * NBUF)]
    )

    @functools.partial(
        pl.kernel,
        mesh=mesh,
        out_type=jax.ShapeDtypeStruct((rows, DIM), jnp.float32),
        scratch_types=scratch,
    )
    def gather_kernel(idx_hbm, table_hbm, out_hbm, *bufs):
        idxs = bufs[0:NBUF]
        rowbufs = bufs[NBUF:2 * NBUF]
        gsems = bufs[2 * NBUF:3 * NBUF]
        ssems = bufs[3 * NBUF:4 * NBUF]

        wid = lax.axis_index("s") * NC + lax.axis_index("c")
        base = wid * rows_per_w

        def fetch(g, b):
            off = base + g * CHUNK
            pltpu.sync_copy(idx_hbm.at[pl.ds(off, CHUNK)], idxs[b])
            pltpu.async_copy(table_hbm.at[idxs[b]], rowbufs[b], gsems[b])

        for b in range(NBUF):
            fetch(b, b)

        def group(i, carry):
            g0 = i * NBUF
            for b in range(NBUF):
                g = g0 + b
                off = base + g * CHUNK
                pltpu.make_async_copy(
                    table_hbm.at[idxs[b]], rowbufs[b], gsems[b]).wait()
                pltpu.async_copy(
                    rowbufs[b], out_hbm.at[pl.ds(off, CHUNK)], ssems[b])
            for b in range(NBUF):
                g = g0 + b
                off = base + g * CHUNK
                pltpu.make_async_copy(
                    rowbufs[b], out_hbm.at[pl.ds(off, CHUNK)],
                    ssems[b]).wait()

                @pl.when(g + NBUF < chunks)
                def _():
                    fetch(g + NBUF, b)

            return carry

        lax.fori_loop(0, groups, group, 0)

    return gather_kernel


def kernel(indices, table):
    batch, seq = indices.shape
    rows = batch * seq
    out = _make_gather(rows)(indices.reshape(rows), table)
    return out.reshape(batch, seq, DIM)
